# baseline (speedup 1.0000x reference)
import functools
import math

import jax
import jax.numpy as jnp
from jax import lax
from jax.experimental import pallas as pl
from jax.experimental.pallas import tpu as pltpu
from jax.experimental.pallas import tpu_sc as plsc

_SPARSITY_K = 128
_ROW_BLOCK = 256
_NC, _NS, _L = 2, 16, 16
_NW = _NC * _NS


def _attn_mlp_kernel(idx_ref, x_ref, wq_ref, wk_ref, wv_ref, wo_ref,
                     wdown_ref, wup_ref, bias_ref, ld_ref, out_ref, *, eps):
    del idx_ref
    g = x_ref[0]
    t_dim, k_dim = g.shape

    def rms(v):
        return v * jax.lax.rsqrt(jnp.mean(v * v, axis=-1, keepdims=True) + eps)

    gn = rms(g)
    dot = functools.partial(jax.lax.dot_general,
                            preferred_element_type=jnp.float32)
    ct = (((1,), (1,)), ((), ()))
    q = dot(gn, wq_ref[...], dimension_numbers=ct)
    k = dot(gn, wk_ref[...], dimension_numbers=ct)
    v = dot(gn, wv_ref[...], dimension_numbers=ct)
    scores = dot(q, k, dimension_numbers=ct)
    trow = jax.lax.broadcasted_iota(jnp.int32, (t_dim, t_dim), 0)
    scol = jax.lax.broadcasted_iota(jnp.int32, (t_dim, t_dim), 1)
    diff = (scol - trow).astype(jnp.float32)
    log_decay = ld_ref[0]
    w = jnp.exp(jnp.maximum(diff - 1.0, 0.0) * log_decay)
    w = jnp.where(scol > trow, w, 0.0)
    scores = scores * w
    retrieved = dot(scores, v, dimension_numbers=(((1,), (0,)), ((), ())))
    g2 = g + dot(retrieved, wo_ref[...], dimension_numbers=ct)
    g2n = rms(g2)
    h = dot(g2n, wdown_ref[...], dimension_numbers=ct) + bias_ref[...]
    h = 0.5 * h * (1.0 + jax.lax.erf(h * (1.0 / math.sqrt(2.0))))
    out_ref[0] = dot(h, wup_ref[...], dimension_numbers=ct)


def _sc_topk_kernel(x_hbm, wo_hbm, out_hbm, xrow, ab, hist, candbuf, wovec,
                    sem_in, *, rows_per_w, v_dim, n_cand):
    wid = lax.axis_index("s") * _NC + lax.axis_index("c")
    base = wid * rows_per_w
    nv = v_dim // _L
    ci = lax.iota(jnp.int32, _L)
    ones_i = jnp.ones((_L,), jnp.int32)
    zeros_i = jnp.zeros((_L,), jnp.int32)

    pltpu.sync_copy(wo_hbm, wovec)
    wo = wovec[...]
    wo_hi = wo + 256

    pltpu.async_copy(x_hbm.at[base], xrow.at[0], sem_in)

    def scan_hist(rk):
        def tb(ii, totv):
            h = hist[pl.ds(ii * _L, _L)]
            return totv + jnp.where(ci == ii, jnp.sum(h, axis=0), 0)
        totv = lax.fori_loop(0, _L, tb, zeros_i)
        rv = lax.rev(totv, (0,))
        cs = plsc.cumsum(rv)
        jj = jnp.max(plsc.all_reduce_ffs(cs >= rk), axis=0)
        iistar = 15 - jj
        selj = ci == jj
        above_v = (jnp.sum(jnp.where(selj, cs, 0), axis=0)
                   - jnp.sum(jnp.where(selj, rv, 0), axis=0))
        h = hist[pl.ds(pl.multiple_of(iistar * _L, _L), _L)]
        hr = lax.rev(h, (0,))
        c2 = plsc.cumsum(hr)
        j2 = jnp.max(plsc.all_reduce_ffs((above_v + c2) >= rk), axis=0)
        selj2 = ci == j2
        bstar = iistar * _L + (15 - j2)
        above_excl = (above_v + jnp.sum(jnp.where(selj2, c2, 0), axis=0)
                      - jnp.sum(jnp.where(selj2, hr, 0), axis=0))
        return bstar, above_excl

    def do_row(r, buf):
        @pl.when(r + 1 < rows_per_w)
        def _():
            pltpu.async_copy(x_hbm.at[base + r + 1], xrow.at[1 - buf], sem_in)
        pltpu.make_async_copy(x_hbm.at[base + r], xrow.at[buf], sem_in).wait()

        def pp(j, _):
            v = xrow[buf, pl.ds(pl.multiple_of(j * _L, _L), _L)]
            a = plsc.bitcast(v, jnp.int32) & 0x7FFFFFFF
            col = ci + j * _L
            excl = (col >= wo) & (col < wo_hi)
            ab[pl.ds(pl.multiple_of(j * _L, _L), _L)] = jnp.where(excl, 0, a)
            return 0
        lax.fori_loop(0, nv, pp, 0, unroll=4)

        rk = jnp.int32(n_cand)
        prefix = jnp.int32(0)
        for sh, w in ((23, 8), (15, 8), (7, 8), (0, 7)):
            def cl(ii, _):
                hist[pl.ds(ii * _L, _L)] = zeros_i
                return 0
            lax.fori_loop(0, _L, cl, 0, unroll=4)

            def hp(j, _, sh=sh, w=w, prefix=prefix):
                a = ab[pl.ds(pl.multiple_of(j * _L, _L), _L)]
                act = lax.shift_right_logical(a, sh + w) == prefix
                b = lax.shift_right_logical(a, sh) & (2 ** w - 1)
                plsc.addupdate_scatter(hist, [b], ones_i, mask=act)
                return 0
            lax.fori_loop(0, nv, hp, 0, unroll=4)
            bstar, above = scan_hist(rk)
            rk = rk - above
            prefix = prefix * (2 ** w) + bstar
        tstar = prefix

        rvec = zeros_i + r

        def cb(j, run):
            a = ab[pl.ds(pl.multiple_of(j * _L, _L), _L)]
            msel = a >= tstar
            mi = jnp.where(msel, 1, 0)
            cs = plsc.cumsum(mi)
            posn = run + cs - mi
            smask = msel & (posn < n_cand)
            plsc.store_scatter(candbuf, [rvec, posn],
                               plsc.bitcast(a, jnp.float32), mask=smask)
            return run + jnp.sum(mi, axis=0)
        lax.fori_loop(0, nv, cb, jnp.int32(0), unroll=4)
        return ()

    def outer(g, _):
        do_row(g * 2, 0)
        do_row(g * 2 + 1, 1)
        return 0
    lax.fori_loop(0, rows_per_w // 2, outer, 0)

    pltpu.sync_copy(candbuf, out_hbm.at[pl.ds(base, rows_per_w)])


def _mask_kernel(idx_ref, x_ref, os_ref, xw_ref, cand_ref, out_ref,
                 *, sparsity_k):
    xr = x_ref[...]
    r_dim, v_dim = xr.shape
    k_dim = os_ref.shape[1]
    m = jnp.max(xr, axis=1, keepdims=True)
    e = jnp.exp(xr - m)
    s = jnp.sum(e, axis=1, keepdims=True)
    p = e / s
    ent = -jnp.sum(p * jnp.log(p + 1e-08), axis=1, keepdims=True)
    scaled = os_ref[...] * ent
    xnw = xw_ref[...] + scaled
    abw = jax.lax.bitcast_convert_type(jnp.abs(xnw), jnp.int32)
    abc = jax.lax.bitcast_convert_type(cand_ref[...], jnp.int32)
    ab = jnp.concatenate([abw, abc], axis=1)

    lo0 = jnp.full((r_dim, 1), -1, jnp.int32)
    hi0 = jnp.full((r_dim, 1), 0x7F800000, jnp.int32)

    def body(_, carry):
        lo, hi = carry
        mid = lo + jax.lax.shift_right_logical(hi - lo, 1)
        cnt = jnp.sum((ab > mid).astype(jnp.int32), axis=1, keepdims=True)
        ge = cnt >= sparsity_k
        return jnp.where(ge, mid, lo), jnp.where(ge, hi, mid)

    lo, hi = jax.lax.fori_loop(0, 31, body, (lo0, hi0))
    abr = jax.lax.bitcast_convert_type(jnp.abs(xr), jnp.int32)
    out_ref[...] = jnp.where(abr >= hi, xr, 0.0)
    wv = idx_ref[0] * k_dim
    out_ref[:, pl.ds(wv, k_dim)] = jnp.where(abw >= hi, xnw, 0.0)


def kernel(x, Wq, Wk, Wv, Wo, decay_logit, out_scale, W_down, W_up, mlp_bias,
           mem_scale, write_scale, read_indices, write_selector):
    b_dim, t_dim, v_dim = x.shape
    k_dim = Wq.shape[0]
    inner = W_down.shape[0]
    eps = 1.1920929e-07
    rows = b_dim * t_dim

    scale = 1.0 / math.sqrt(k_dim)
    wq2 = Wq * scale
    wo2 = Wo * (out_scale * mem_scale[0])
    wup2 = W_up * (write_scale / (math.sqrt(k_dim) * math.log(v_dim)))
    log_decay = jnp.log(jax.nn.sigmoid(decay_logit)).reshape(1)
    ro_blk = (read_indices[0].astype(jnp.int32) // k_dim).reshape(1)
    wo_col = jnp.argmax(write_selector[0]).astype(jnp.int32)
    wo_blk = (wo_col // k_dim).reshape(1)

    x2 = x.reshape(rows, v_dim)

    rows_per_w = rows // _NW
    mesh = plsc.VectorSubcoreMesh(core_axis_name="c", subcore_axis_name="s")
    sc_topk = functools.partial(
        pl.kernel,
        out_type=jax.ShapeDtypeStruct((rows, _SPARSITY_K), jnp.float32),
        mesh=mesh,
        scratch_types=[
            pltpu.VMEM((2, v_dim), jnp.float32),
            pltpu.VMEM((v_dim,), jnp.int32),
            pltpu.VMEM((256,), jnp.int32),
            pltpu.VMEM((rows_per_w, _SPARSITY_K), jnp.float32),
            pltpu.VMEM((_L,), jnp.int32),
            pltpu.SemaphoreType.DMA,
        ],
        compiler_params=pltpu.CompilerParams(needs_layout_passes=False),
    )(functools.partial(_sc_topk_kernel, rows_per_w=rows_per_w, v_dim=v_dim,
                        n_cand=_SPARSITY_K))
    cands = sc_topk(x2, jnp.full((_L,), wo_col, jnp.int32))

    grid_a = pltpu.PrefetchScalarGridSpec(
        num_scalar_prefetch=1,
        grid=(b_dim,),
        in_specs=[
            pl.BlockSpec((1, t_dim, k_dim), lambda b, idx: (b, 0, idx[0])),
            pl.BlockSpec((k_dim, k_dim), lambda b, idx: (0, 0)),
            pl.BlockSpec((k_dim, k_dim), lambda b, idx: (0, 0)),
            pl.BlockSpec((k_dim, k_dim), lambda b, idx: (0, 0)),
            pl.BlockSpec((k_dim, k_dim), lambda b, idx: (0, 0)),
            pl.BlockSpec((inner, k_dim), lambda b, idx: (0, 0)),
            pl.BlockSpec((k_dim, inner), lambda b, idx: (0, 0)),
            pl.BlockSpec((1, inner), lambda b, idx: (0, 0)),
            pl.BlockSpec(memory_space=pltpu.SMEM),
        ],
        out_specs=pl.BlockSpec((1, t_dim, k_dim), lambda b, idx: (b, 0, 0)),
    )
    out_small = pl.pallas_call(
        functools.partial(_attn_mlp_kernel, eps=eps),
        grid_spec=grid_a,
        out_shape=jax.ShapeDtypeStruct((b_dim, t_dim, k_dim), jnp.float32),
    )(ro_blk, x, wq2, Wk, Wv, wo2, W_down, wup2,
      mlp_bias.reshape(1, inner), log_decay)

    rblk = min(_ROW_BLOCK, rows)
    os2 = out_small.reshape(rows, k_dim)
    grid_b = pltpu.PrefetchScalarGridSpec(
        num_scalar_prefetch=1,
        grid=(rows // rblk,),
        in_specs=[
            pl.BlockSpec((rblk, v_dim), lambda i, idx: (i, 0)),
            pl.BlockSpec((rblk, k_dim), lambda i, idx: (i, 0)),
            pl.BlockSpec((rblk, k_dim), lambda i, idx: (i, idx[0])),
            pl.BlockSpec((rblk, _SPARSITY_K), lambda i, idx: (i, 0)),
        ],
        out_specs=pl.BlockSpec((rblk, v_dim), lambda i, idx: (i, 0)),
    )
    out = pl.pallas_call(
        functools.partial(_mask_kernel, sparsity_k=_SPARSITY_K),
        grid_spec=grid_b,
        out_shape=jax.ShapeDtypeStruct((rows, v_dim), jnp.float32),
    )(wo_blk, x2, os2, x2, cands)
    return out.reshape(b_dim, t_dim, v_dim)


# speedup vs baseline: 2.8298x; 2.8298x over previous
import functools
import math

import jax
import jax.numpy as jnp
from jax import lax
from jax.experimental import pallas as pl
from jax.experimental.pallas import tpu as pltpu
from jax.experimental.pallas import tpu_sc as plsc

_SPARSITY_K = 128
_ROW_BLOCK = 256
_NC, _NS, _L = 2, 16, 16
_NW = _NC * _NS


def _attn_mlp_kernel(idx_ref, x_ref, wq_ref, wk_ref, wv_ref, wo_ref,
                     wdown_ref, wup_ref, bias_ref, ld_ref, out_ref, *, eps):
    del idx_ref
    g = x_ref[0]
    t_dim, k_dim = g.shape

    def rms(v):
        return v * jax.lax.rsqrt(jnp.mean(v * v, axis=-1, keepdims=True) + eps)

    gn = rms(g)
    dot = functools.partial(jax.lax.dot_general,
                            preferred_element_type=jnp.float32)
    ct = (((1,), (1,)), ((), ()))
    q = dot(gn, wq_ref[...], dimension_numbers=ct)
    k = dot(gn, wk_ref[...], dimension_numbers=ct)
    v = dot(gn, wv_ref[...], dimension_numbers=ct)
    scores = dot(q, k, dimension_numbers=ct)
    trow = jax.lax.broadcasted_iota(jnp.int32, (t_dim, t_dim), 0)
    scol = jax.lax.broadcasted_iota(jnp.int32, (t_dim, t_dim), 1)
    diff = (scol - trow).astype(jnp.float32)
    log_decay = ld_ref[0]
    w = jnp.exp(jnp.maximum(diff - 1.0, 0.0) * log_decay)
    w = jnp.where(scol > trow, w, 0.0)
    scores = scores * w
    retrieved = dot(scores, v, dimension_numbers=(((1,), (0,)), ((), ())))
    g2 = g + dot(retrieved, wo_ref[...], dimension_numbers=ct)
    g2n = rms(g2)
    h = dot(g2n, wdown_ref[...], dimension_numbers=ct) + bias_ref[...]
    h = 0.5 * h * (1.0 + jax.lax.erf(h * (1.0 / math.sqrt(2.0))))
    out_ref[0] = dot(h, wup_ref[...], dimension_numbers=ct)


def _sc_topk_kernel(x_hbm, wo_hbm, out_hbm, xrow, ab, hist, candbuf, wovec,
                    sem_in, *, rows_per_w, v_dim, n_cand, row_base):
    wid = lax.axis_index("s") * _NC + lax.axis_index("c")
    obase = wid * rows_per_w
    base = row_base + obase
    nv = v_dim // _L
    ci = lax.iota(jnp.int32, _L)
    ones_i = jnp.ones((_L,), jnp.int32)
    zeros_i = jnp.zeros((_L,), jnp.int32)

    pltpu.sync_copy(wo_hbm, wovec)
    wo = wovec[...]
    wo_hi = wo + 256

    pltpu.async_copy(x_hbm.at[base], xrow.at[0], sem_in)

    def scan_hist(rk):
        def tb(ii, totv):
            h = hist[pl.ds(ii * _L, _L)]
            return totv + jnp.where(ci == ii, jnp.sum(h, axis=0), 0)
        totv = lax.fori_loop(0, _L, tb, zeros_i)
        rv = lax.rev(totv, (0,))
        cs = plsc.cumsum(rv)
        jj = jnp.max(plsc.all_reduce_ffs(cs >= rk), axis=0)
        iistar = 15 - jj
        selj = ci == jj
        above_v = (jnp.sum(jnp.where(selj, cs, 0), axis=0)
                   - jnp.sum(jnp.where(selj, rv, 0), axis=0))
        h = hist[pl.ds(pl.multiple_of(iistar * _L, _L), _L)]
        hr = lax.rev(h, (0,))
        c2 = plsc.cumsum(hr)
        j2 = jnp.max(plsc.all_reduce_ffs((above_v + c2) >= rk), axis=0)
        selj2 = ci == j2
        bstar = iistar * _L + (15 - j2)
        above_excl = (above_v + jnp.sum(jnp.where(selj2, c2, 0), axis=0)
                      - jnp.sum(jnp.where(selj2, hr, 0), axis=0))
        return bstar, above_excl

    def do_row(r, buf):
        @pl.when(r + 1 < rows_per_w)
        def _():
            pltpu.async_copy(x_hbm.at[base + r + 1], xrow.at[1 - buf], sem_in)
        pltpu.make_async_copy(x_hbm.at[base + r], xrow.at[buf], sem_in).wait()

        def pp(j, _):
            v = xrow[buf, pl.ds(pl.multiple_of(j * _L, _L), _L)]
            a = plsc.bitcast(v, jnp.int32) & 0x7FFFFFFF
            col = ci + j * _L
            excl = (col >= wo) & (col < wo_hi)
            ab[pl.ds(pl.multiple_of(j * _L, _L), _L)] = jnp.where(excl, 0, a)
            return 0
        lax.fori_loop(0, nv, pp, 0, unroll=4)

        rk = jnp.int32(n_cand)
        prefix = jnp.int32(0)
        for sh, w in ((23, 8), (15, 8), (7, 8), (0, 7)):
            def cl(ii, _):
                hist[pl.ds(ii * _L, _L)] = zeros_i
                return 0
            lax.fori_loop(0, _L, cl, 0, unroll=4)

            def hp(j, _, sh=sh, w=w, prefix=prefix):
                a = ab[pl.ds(pl.multiple_of(j * _L, _L), _L)]
                act = lax.shift_right_logical(a, sh + w) == prefix
                b = lax.shift_right_logical(a, sh) & (2 ** w - 1)
                plsc.addupdate_scatter(hist, [b], ones_i, mask=act)
                return 0
            lax.fori_loop(0, nv, hp, 0, unroll=4)
            bstar, above = scan_hist(rk)
            rk = rk - above
            prefix = prefix * (2 ** w) + bstar
        tstar = prefix

        rvec = zeros_i + r

        def cb(j, run):
            a = ab[pl.ds(pl.multiple_of(j * _L, _L), _L)]
            msel = a >= tstar
            mi = jnp.where(msel, 1, 0)
            cs = plsc.cumsum(mi)
            posn = run + cs - mi
            smask = msel & (posn < n_cand)
            plsc.store_scatter(candbuf, [rvec, posn],
                               plsc.bitcast(a, jnp.float32), mask=smask)
            return run + jnp.sum(mi, axis=0)
        lax.fori_loop(0, nv, cb, jnp.int32(0), unroll=4)
        return ()

    def outer(g, _):
        do_row(g * 2, 0)
        do_row(g * 2 + 1, 1)
        return 0
    lax.fori_loop(0, rows_per_w // 2, outer, 0)

    pltpu.sync_copy(candbuf, out_hbm.at[pl.ds(obase, rows_per_w)])


def _mask_kernel(idx_ref, x_ref, os_ref, xw_ref, cand_ref, out_ref,
                 *, sparsity_k, split_blk):
    xr = x_ref[...]
    r_dim, v_dim = xr.shape
    k_dim = os_ref.shape[1]
    m = jnp.max(xr, axis=1, keepdims=True)
    e = jnp.exp(xr - m)
    s = jnp.sum(e, axis=1, keepdims=True)
    p = e / s
    ent = -jnp.sum(p * jnp.log(p + 1e-08), axis=1, keepdims=True)
    scaled = os_ref[...] * ent
    xw = xw_ref[...]
    xnw = xw + scaled
    abw = jax.lax.bitcast_convert_type(jnp.abs(xnw), jnp.int32)
    abwo = jax.lax.bitcast_convert_type(jnp.abs(xw), jnp.int32)
    abr = jax.lax.bitcast_convert_type(jnp.abs(xr), jnp.int32)

    def count3(a_big, a_old, a_new, mid):
        c1 = jnp.sum((a_big > mid).astype(jnp.int32), axis=1, keepdims=True)
        c2 = jnp.sum((a_old > mid).astype(jnp.int32), axis=1, keepdims=True)
        c3 = jnp.sum((a_new > mid).astype(jnp.int32), axis=1, keepdims=True)
        return c1 - c2 + c3

    def full_path():
        abh = jax.lax.shift_right_logical(abr, 16).astype(jnp.int16)
        aboh = jax.lax.shift_right_logical(abwo, 16).astype(jnp.int16)
        abnh = jax.lax.shift_right_logical(abw, 16).astype(jnp.int16)
        lo0 = jnp.full((r_dim, 1), -1, jnp.int32)
        hi0 = jnp.full((r_dim, 1), 0x7FFF, jnp.int32)

        def b1(_, carry):
            lo, hi = carry
            mid = lo + jax.lax.shift_right_logical(hi - lo, 1)
            cnt = count3(abh, aboh, abnh, mid.astype(jnp.int16))
            ge = cnt >= sparsity_k
            return jnp.where(ge, mid, lo), jnp.where(ge, hi, mid)

        _, hi1 = jax.lax.fori_loop(0, 15, b1, (lo0, hi0))
        base = hi1 * 65536
        lo0b = base - 1
        hi0b = base + 65535

        def b2(_, carry):
            lo, hi = carry
            mid = lo + jax.lax.shift_right_logical(hi - lo, 1)
            ge = count3(abr, abwo, abw, mid) >= sparsity_k
            return jnp.where(ge, mid, lo), jnp.where(ge, hi, mid)

        _, hi2 = jax.lax.fori_loop(0, 16, b2, (lo0b, hi0b))
        return hi2

    def merge_path():
        abc = jax.lax.bitcast_convert_type(cand_ref[...], jnp.int32)
        ab = jnp.concatenate([abw, abc], axis=1)
        lo0 = jnp.full((r_dim, 1), -1, jnp.int32)
        hi0 = jnp.full((r_dim, 1), 0x7F800000, jnp.int32)

        def body(_, carry):
            lo, hi = carry
            mid = lo + jax.lax.shift_right_logical(hi - lo, 1)
            cnt = jnp.sum((ab > mid).astype(jnp.int32), axis=1, keepdims=True)
            ge = cnt >= sparsity_k
            return jnp.where(ge, mid, lo), jnp.where(ge, hi, mid)

        _, hi = jax.lax.fori_loop(0, 31, body, (lo0, hi0))
        return hi

    hi = jax.lax.cond(pl.program_id(0) < split_blk, full_path, merge_path)
    out_ref[...] = jnp.where(abr >= hi, xr, 0.0)
    wv = idx_ref[0] * k_dim
    out_ref[:, pl.ds(wv, k_dim)] = jnp.where(abw >= hi, xnw, 0.0)


def kernel(x, Wq, Wk, Wv, Wo, decay_logit, out_scale, W_down, W_up, mlp_bias,
           mem_scale, write_scale, read_indices, write_selector):
    b_dim, t_dim, v_dim = x.shape
    k_dim = Wq.shape[0]
    inner = W_down.shape[0]
    eps = 1.1920929e-07
    rows = b_dim * t_dim

    scale = 1.0 / math.sqrt(k_dim)
    wq2 = Wq * scale
    wo2 = Wo * (out_scale * mem_scale[0])
    wup2 = W_up * (write_scale / (math.sqrt(k_dim) * math.log(v_dim)))
    log_decay = jnp.log(jax.nn.sigmoid(decay_logit)).reshape(1)
    ro_blk = (read_indices[0].astype(jnp.int32) // k_dim).reshape(1)
    wo_col = jnp.argmax(write_selector[0]).astype(jnp.int32)
    wo_blk = (wo_col // k_dim).reshape(1)

    x2 = x.reshape(rows, v_dim)

    rblk = min(_ROW_BLOCK, rows)
    n_blk = rows // rblk
    split_blk = max(n_blk - 1, 0)
    sc_rows = rows - split_blk * rblk
    row_base = split_blk * rblk

    rows_per_w = sc_rows // _NW
    mesh = plsc.VectorSubcoreMesh(core_axis_name="c", subcore_axis_name="s")
    sc_topk = functools.partial(
        pl.kernel,
        out_type=jax.ShapeDtypeStruct((sc_rows, _SPARSITY_K), jnp.float32),
        mesh=mesh,
        scratch_types=[
            pltpu.VMEM((2, v_dim), jnp.float32),
            pltpu.VMEM((v_dim,), jnp.int32),
            pltpu.VMEM((256,), jnp.int32),
            pltpu.VMEM((rows_per_w, _SPARSITY_K), jnp.float32),
            pltpu.VMEM((_L,), jnp.int32),
            pltpu.SemaphoreType.DMA,
        ],
        compiler_params=pltpu.CompilerParams(needs_layout_passes=False),
    )(functools.partial(_sc_topk_kernel, rows_per_w=rows_per_w, v_dim=v_dim,
                        n_cand=_SPARSITY_K, row_base=row_base))
    cands = sc_topk(x2, jnp.full((_L,), wo_col, jnp.int32))

    grid_a = pltpu.PrefetchScalarGridSpec(
        num_scalar_prefetch=1,
        grid=(b_dim,),
        in_specs=[
            pl.BlockSpec((1, t_dim, k_dim), lambda b, idx: (b, 0, idx[0])),
            pl.BlockSpec((k_dim, k_dim), lambda b, idx: (0, 0)),
            pl.BlockSpec((k_dim, k_dim), lambda b, idx: (0, 0)),
            pl.BlockSpec((k_dim, k_dim), lambda b, idx: (0, 0)),
            pl.BlockSpec((k_dim, k_dim), lambda b, idx: (0, 0)),
            pl.BlockSpec((inner, k_dim), lambda b, idx: (0, 0)),
            pl.BlockSpec((k_dim, inner), lambda b, idx: (0, 0)),
            pl.BlockSpec((1, inner), lambda b, idx: (0, 0)),
            pl.BlockSpec(memory_space=pltpu.SMEM),
        ],
        out_specs=pl.BlockSpec((1, t_dim, k_dim), lambda b, idx: (b, 0, 0)),
    )
    out_small = pl.pallas_call(
        functools.partial(_attn_mlp_kernel, eps=eps),
        grid_spec=grid_a,
        out_shape=jax.ShapeDtypeStruct((b_dim, t_dim, k_dim), jnp.float32),
    )(ro_blk, x, wq2, Wk, Wv, wo2, W_down, wup2,
      mlp_bias.reshape(1, inner), log_decay)

    os2 = out_small.reshape(rows, k_dim)
    grid_b = pltpu.PrefetchScalarGridSpec(
        num_scalar_prefetch=1,
        grid=(n_blk,),
        in_specs=[
            pl.BlockSpec((rblk, v_dim), lambda i, idx: (i, 0)),
            pl.BlockSpec((rblk, k_dim), lambda i, idx: (i, 0)),
            pl.BlockSpec((rblk, k_dim), lambda i, idx: (i, idx[0])),
            pl.BlockSpec((rblk, _SPARSITY_K), lambda i, idx: (0, 0)),
        ],
        out_specs=pl.BlockSpec((rblk, v_dim), lambda i, idx: (i, 0)),
    )
    out = pl.pallas_call(
        functools.partial(_mask_kernel, sparsity_k=_SPARSITY_K,
                          split_blk=split_blk),
        grid_spec=grid_b,
        out_shape=jax.ShapeDtypeStruct((rows, v_dim), jnp.float32),
    )(wo_blk, x2, os2, x2, cands)
    return out.reshape(b_dim, t_dim, v_dim)


# speedup vs baseline: 3.1094x; 1.0988x over previous
import functools
import math

import jax
import jax.numpy as jnp
from jax import lax
from jax.experimental import pallas as pl
from jax.experimental.pallas import tpu as pltpu
from jax.experimental.pallas import tpu_sc as plsc

_SPARSITY_K = 128
_ROW_BLOCK = 256
_NC, _NS, _L = 2, 16, 16
_NW = _NC * _NS


def _attn_mlp_kernel(idx_ref, x_ref, wq_ref, wk_ref, wv_ref, wo_ref,
                     wdown_ref, wup_ref, bias_ref, ld_ref, out_ref, *, eps):
    del idx_ref
    g = x_ref[0]
    t_dim, k_dim = g.shape

    def rms(v):
        return v * jax.lax.rsqrt(jnp.mean(v * v, axis=-1, keepdims=True) + eps)

    gn = rms(g)
    dot = functools.partial(jax.lax.dot_general,
                            preferred_element_type=jnp.float32)
    ct = (((1,), (1,)), ((), ()))
    q = dot(gn, wq_ref[...], dimension_numbers=ct)
    k = dot(gn, wk_ref[...], dimension_numbers=ct)
    v = dot(gn, wv_ref[...], dimension_numbers=ct)
    scores = dot(q, k, dimension_numbers=ct)
    trow = jax.lax.broadcasted_iota(jnp.int32, (t_dim, t_dim), 0)
    scol = jax.lax.broadcasted_iota(jnp.int32, (t_dim, t_dim), 1)
    diff = (scol - trow).astype(jnp.float32)
    log_decay = ld_ref[0]
    w = jnp.exp(jnp.maximum(diff - 1.0, 0.0) * log_decay)
    w = jnp.where(scol > trow, w, 0.0)
    scores = scores * w
    retrieved = dot(scores, v, dimension_numbers=(((1,), (0,)), ((), ())))
    g2 = g + dot(retrieved, wo_ref[...], dimension_numbers=ct)
    g2n = rms(g2)
    h = dot(g2n, wdown_ref[...], dimension_numbers=ct) + bias_ref[...]
    h = 0.5 * h * (1.0 + jax.lax.erf(h * (1.0 / math.sqrt(2.0))))
    out_ref[0] = dot(h, wup_ref[...], dimension_numbers=ct)


def _sc_topk_kernel(x_hbm, wo_hbm, out_hbm, xrow, ab, hist, candbuf, wovec,
                    sem_in, *, rows_per_w, v_dim, n_cand, row_base):
    wid = lax.axis_index("s") * _NC + lax.axis_index("c")
    obase = wid * rows_per_w
    base = row_base + obase
    nv = v_dim // _L
    ci = lax.iota(jnp.int32, _L)
    ones_i = jnp.ones((_L,), jnp.int32)
    zeros_i = jnp.zeros((_L,), jnp.int32)

    pltpu.sync_copy(wo_hbm, wovec)
    wo = wovec[...]
    wo_hi = wo + 256

    pltpu.async_copy(x_hbm.at[base], xrow.at[0], sem_in)

    def scan_hist(rk):
        def tb(ii, totv):
            h = hist[pl.ds(ii * _L, _L)]
            return totv + jnp.where(ci == ii, jnp.sum(h, axis=0), 0)
        totv = lax.fori_loop(0, _L, tb, zeros_i)
        rv = lax.rev(totv, (0,))
        cs = plsc.cumsum(rv)
        jj = jnp.max(plsc.all_reduce_ffs(cs >= rk), axis=0)
        iistar = 15 - jj
        selj = ci == jj
        above_v = (jnp.sum(jnp.where(selj, cs, 0), axis=0)
                   - jnp.sum(jnp.where(selj, rv, 0), axis=0))
        h = hist[pl.ds(pl.multiple_of(iistar * _L, _L), _L)]
        hr = lax.rev(h, (0,))
        c2 = plsc.cumsum(hr)
        j2 = jnp.max(plsc.all_reduce_ffs((above_v + c2) >= rk), axis=0)
        selj2 = ci == j2
        bstar = iistar * _L + (15 - j2)
        above_excl = (above_v + jnp.sum(jnp.where(selj2, c2, 0), axis=0)
                      - jnp.sum(jnp.where(selj2, hr, 0), axis=0))
        return bstar, above_excl

    def do_row(r, buf):
        @pl.when(r + 1 < rows_per_w)
        def _():
            pltpu.async_copy(x_hbm.at[base + r + 1], xrow.at[1 - buf], sem_in)
        pltpu.make_async_copy(x_hbm.at[base + r], xrow.at[buf], sem_in).wait()

        def pp(j, _):
            v = xrow[buf, pl.ds(pl.multiple_of(j * _L, _L), _L)]
            a = plsc.bitcast(v, jnp.int32) & 0x7FFFFFFF
            col = ci + j * _L
            excl = (col >= wo) & (col < wo_hi)
            ab[pl.ds(pl.multiple_of(j * _L, _L), _L)] = jnp.where(excl, 0, a)
            return 0
        lax.fori_loop(0, nv, pp, 0, unroll=4)

        rk = jnp.int32(n_cand)
        prefix = jnp.int32(0)
        for sh, w in ((23, 8), (15, 8), (7, 8), (0, 7)):
            def cl(ii, _):
                hist[pl.ds(ii * _L, _L)] = zeros_i
                return 0
            lax.fori_loop(0, _L, cl, 0, unroll=4)

            def hp(j, _, sh=sh, w=w, prefix=prefix):
                a = ab[pl.ds(pl.multiple_of(j * _L, _L), _L)]
                act = lax.shift_right_logical(a, sh + w) == prefix
                b = lax.shift_right_logical(a, sh) & (2 ** w - 1)
                plsc.addupdate_scatter(hist, [b], ones_i, mask=act)
                return 0
            lax.fori_loop(0, nv, hp, 0, unroll=4)
            bstar, above = scan_hist(rk)
            rk = rk - above
            prefix = prefix * (2 ** w) + bstar
        tstar = prefix

        rvec = zeros_i + r

        def cb(j, run):
            a = ab[pl.ds(pl.multiple_of(j * _L, _L), _L)]
            msel = a >= tstar
            mi = jnp.where(msel, 1, 0)
            cs = plsc.cumsum(mi)
            posn = run + cs - mi
            smask = msel & (posn < n_cand)
            plsc.store_scatter(candbuf, [rvec, posn],
                               plsc.bitcast(a, jnp.float32), mask=smask)
            return run + jnp.sum(mi, axis=0)
        lax.fori_loop(0, nv, cb, jnp.int32(0), unroll=4)
        return ()

    def outer(g, _):
        do_row(g * 2, 0)
        do_row(g * 2 + 1, 1)
        return 0
    lax.fori_loop(0, rows_per_w // 2, outer, 0)

    pltpu.sync_copy(candbuf, out_hbm.at[pl.ds(obase, rows_per_w)])


def _mask_kernel(idx_ref, x_ref, os_ref, xw_ref, cand_ref, out_ref,
                 *, sparsity_k, split_blk):
    xr = x_ref[...]
    r_dim, v_dim = xr.shape
    k_dim = os_ref.shape[1]
    m = jnp.max(xr, axis=1, keepdims=True)
    e = jnp.exp(xr - m)
    s = jnp.sum(e, axis=1, keepdims=True)
    p = e / s
    ent = -jnp.sum(p * jnp.log(p + 1e-08), axis=1, keepdims=True)
    scaled = os_ref[...] * ent
    xw = xw_ref[...]
    xnw = xw + scaled
    abw = jax.lax.bitcast_convert_type(jnp.abs(xnw), jnp.int32)
    abwo = jax.lax.bitcast_convert_type(jnp.abs(xw), jnp.int32)
    abr = jax.lax.bitcast_convert_type(jnp.abs(xr), jnp.int32)

    def count3(a_big, a_old, a_new, mid):
        c1 = jnp.sum((a_big > mid).astype(jnp.int32), axis=1, keepdims=True)
        c2 = jnp.sum((a_old > mid).astype(jnp.int32), axis=1, keepdims=True)
        c3 = jnp.sum((a_new > mid).astype(jnp.int32), axis=1, keepdims=True)
        return c1 - c2 + c3

    def full_path():
        lo0 = jnp.full((r_dim, 1), -1, jnp.int32)
        hi0 = jnp.full((r_dim, 1), 0x7F800000, jnp.int32)

        def b1(_, carry):
            lo, hi = carry
            mid = lo + jax.lax.shift_right_logical(hi - lo, 1)
            ge = count3(abr, abwo, abw, mid) >= sparsity_k
            return jnp.where(ge, mid, lo), jnp.where(ge, hi, mid)

        _, hi2 = jax.lax.fori_loop(0, 31, b1, (lo0, hi0))
        return hi2

    def merge_path():
        abc = jax.lax.bitcast_convert_type(cand_ref[...], jnp.int32)
        ab = jnp.concatenate([abw, abc], axis=1)
        lo0 = jnp.full((r_dim, 1), -1, jnp.int32)
        hi0 = jnp.full((r_dim, 1), 0x7F800000, jnp.int32)

        def body(_, carry):
            lo, hi = carry
            mid = lo + jax.lax.shift_right_logical(hi - lo, 1)
            cnt = jnp.sum((ab > mid).astype(jnp.int32), axis=1, keepdims=True)
            ge = cnt >= sparsity_k
            return jnp.where(ge, mid, lo), jnp.where(ge, hi, mid)

        _, hi = jax.lax.fori_loop(0, 31, body, (lo0, hi0))
        return hi

    hi = jax.lax.cond(pl.program_id(0) < split_blk, full_path, merge_path)
    out_ref[...] = jnp.where(abr >= hi, xr, 0.0)
    wv = idx_ref[0] * k_dim
    out_ref[:, pl.ds(wv, k_dim)] = jnp.where(abw >= hi, xnw, 0.0)


def kernel(x, Wq, Wk, Wv, Wo, decay_logit, out_scale, W_down, W_up, mlp_bias,
           mem_scale, write_scale, read_indices, write_selector):
    b_dim, t_dim, v_dim = x.shape
    k_dim = Wq.shape[0]
    inner = W_down.shape[0]
    eps = 1.1920929e-07
    rows = b_dim * t_dim

    scale = 1.0 / math.sqrt(k_dim)
    wq2 = Wq * scale
    wo2 = Wo * (out_scale * mem_scale[0])
    wup2 = W_up * (write_scale / (math.sqrt(k_dim) * math.log(v_dim)))
    log_decay = jnp.log(jax.nn.sigmoid(decay_logit)).reshape(1)
    ro_blk = (read_indices[0].astype(jnp.int32) // k_dim).reshape(1)
    wo_col = jnp.argmax(write_selector[0]).astype(jnp.int32)
    wo_blk = (wo_col // k_dim).reshape(1)

    x2 = x.reshape(rows, v_dim)

    rblk = min(_ROW_BLOCK, rows)
    n_blk = rows // rblk
    split_blk = max(n_blk - 1, 0)
    sc_rows = rows - split_blk * rblk
    row_base = split_blk * rblk

    rows_per_w = sc_rows // _NW
    mesh = plsc.VectorSubcoreMesh(core_axis_name="c", subcore_axis_name="s")
    sc_topk = functools.partial(
        pl.kernel,
        out_type=jax.ShapeDtypeStruct((sc_rows, _SPARSITY_K), jnp.float32),
        mesh=mesh,
        scratch_types=[
            pltpu.VMEM((2, v_dim), jnp.float32),
            pltpu.VMEM((v_dim,), jnp.int32),
            pltpu.VMEM((256,), jnp.int32),
            pltpu.VMEM((rows_per_w, _SPARSITY_K), jnp.float32),
            pltpu.VMEM((_L,), jnp.int32),
            pltpu.SemaphoreType.DMA,
        ],
        compiler_params=pltpu.CompilerParams(needs_layout_passes=False),
    )(functools.partial(_sc_topk_kernel, rows_per_w=rows_per_w, v_dim=v_dim,
                        n_cand=_SPARSITY_K, row_base=row_base))
    cands = sc_topk(x2, jnp.full((_L,), wo_col, jnp.int32))

    grid_a = pltpu.PrefetchScalarGridSpec(
        num_scalar_prefetch=1,
        grid=(b_dim,),
        in_specs=[
            pl.BlockSpec((1, t_dim, k_dim), lambda b, idx: (b, 0, idx[0])),
            pl.BlockSpec((k_dim, k_dim), lambda b, idx: (0, 0)),
            pl.BlockSpec((k_dim, k_dim), lambda b, idx: (0, 0)),
            pl.BlockSpec((k_dim, k_dim), lambda b, idx: (0, 0)),
            pl.BlockSpec((k_dim, k_dim), lambda b, idx: (0, 0)),
            pl.BlockSpec((inner, k_dim), lambda b, idx: (0, 0)),
            pl.BlockSpec((k_dim, inner), lambda b, idx: (0, 0)),
            pl.BlockSpec((1, inner), lambda b, idx: (0, 0)),
            pl.BlockSpec(memory_space=pltpu.SMEM),
        ],
        out_specs=pl.BlockSpec((1, t_dim, k_dim), lambda b, idx: (b, 0, 0)),
    )
    out_small = pl.pallas_call(
        functools.partial(_attn_mlp_kernel, eps=eps),
        grid_spec=grid_a,
        out_shape=jax.ShapeDtypeStruct((b_dim, t_dim, k_dim), jnp.float32),
    )(ro_blk, x, wq2, Wk, Wv, wo2, W_down, wup2,
      mlp_bias.reshape(1, inner), log_decay)

    os2 = out_small.reshape(rows, k_dim)
    grid_b = pltpu.PrefetchScalarGridSpec(
        num_scalar_prefetch=1,
        grid=(n_blk,),
        in_specs=[
            pl.BlockSpec((rblk, v_dim), lambda i, idx: (i, 0)),
            pl.BlockSpec((rblk, k_dim), lambda i, idx: (i, 0)),
            pl.BlockSpec((rblk, k_dim), lambda i, idx: (i, idx[0])),
            pl.BlockSpec((rblk, _SPARSITY_K), lambda i, idx: (0, 0)),
        ],
        out_specs=pl.BlockSpec((rblk, v_dim), lambda i, idx: (i, 0)),
    )
    out = pl.pallas_call(
        functools.partial(_mask_kernel, sparsity_k=_SPARSITY_K,
                          split_blk=split_blk),
        grid_spec=grid_b,
        out_shape=jax.ShapeDtypeStruct((rows, v_dim), jnp.float32),
    )(wo_blk, x2, os2, x2, cands)
    return out.reshape(b_dim, t_dim, v_dim)


# speedup vs baseline: 4.6103x; 1.4827x over previous
import functools
import math

import jax
import jax.numpy as jnp
from jax import lax
from jax.experimental import pallas as pl
from jax.experimental.pallas import tpu as pltpu
from jax.experimental.pallas import tpu_sc as plsc

_SPARSITY_K = 128
_ROW_BLOCK = 256
_NC, _NS, _L = 2, 16, 16
_NW = _NC * _NS


def _attn_mlp_kernel(idx_ref, x_ref, wq_ref, wk_ref, wv_ref, wo_ref,
                     wdown_ref, wup_ref, bias_ref, ld_ref, out_ref, *, eps):
    del idx_ref
    g = x_ref[0]
    t_dim, k_dim = g.shape

    def rms(v):
        return v * jax.lax.rsqrt(jnp.mean(v * v, axis=-1, keepdims=True) + eps)

    gn = rms(g)
    dot = functools.partial(jax.lax.dot_general,
                            preferred_element_type=jnp.float32)
    ct = (((1,), (1,)), ((), ()))
    q = dot(gn, wq_ref[...], dimension_numbers=ct)
    k = dot(gn, wk_ref[...], dimension_numbers=ct)
    v = dot(gn, wv_ref[...], dimension_numbers=ct)
    scores = dot(q, k, dimension_numbers=ct)
    trow = jax.lax.broadcasted_iota(jnp.int32, (t_dim, t_dim), 0)
    scol = jax.lax.broadcasted_iota(jnp.int32, (t_dim, t_dim), 1)
    diff = (scol - trow).astype(jnp.float32)
    log_decay = ld_ref[0]
    w = jnp.exp(jnp.maximum(diff - 1.0, 0.0) * log_decay)
    w = jnp.where(scol > trow, w, 0.0)
    scores = scores * w
    retrieved = dot(scores, v, dimension_numbers=(((1,), (0,)), ((), ())))
    g2 = g + dot(retrieved, wo_ref[...], dimension_numbers=ct)
    g2n = rms(g2)
    h = dot(g2n, wdown_ref[...], dimension_numbers=ct) + bias_ref[...]
    h = 0.5 * h * (1.0 + jax.lax.erf(h * (1.0 / math.sqrt(2.0))))
    out_ref[0] = dot(h, wup_ref[...], dimension_numbers=ct)


def _sc_topk_kernel(x_hbm, wo_hbm, out_hbm, xrow, ab, hist, candbuf, wovec,
                    sem_in, *, rows_per_w, v_dim, n_cand, row_base):
    wid = lax.axis_index("s") * _NC + lax.axis_index("c")
    obase = wid * rows_per_w
    base = row_base + obase
    nv = v_dim // _L
    ci = lax.iota(jnp.int32, _L)
    ones_i = jnp.ones((_L,), jnp.int32)
    zeros_i = jnp.zeros((_L,), jnp.int32)

    pltpu.sync_copy(wo_hbm, wovec)
    wo = wovec[...]
    wo_hi = wo + 256

    pltpu.async_copy(x_hbm.at[base], xrow.at[0], sem_in)

    def scan_hist(rk):
        def tb(ii, totv):
            h = hist[pl.ds(ii * _L, _L)]
            return totv + jnp.where(ci == ii, jnp.sum(h, axis=0), 0)
        totv = lax.fori_loop(0, _L, tb, zeros_i)
        rv = lax.rev(totv, (0,))
        cs = plsc.cumsum(rv)
        jj = jnp.max(plsc.all_reduce_ffs(cs >= rk), axis=0)
        iistar = 15 - jj
        selj = ci == jj
        above_v = (jnp.sum(jnp.where(selj, cs, 0), axis=0)
                   - jnp.sum(jnp.where(selj, rv, 0), axis=0))
        h = hist[pl.ds(pl.multiple_of(iistar * _L, _L), _L)]
        hr = lax.rev(h, (0,))
        c2 = plsc.cumsum(hr)
        j2 = jnp.max(plsc.all_reduce_ffs((above_v + c2) >= rk), axis=0)
        selj2 = ci == j2
        bstar = iistar * _L + (15 - j2)
        above_excl = (above_v + jnp.sum(jnp.where(selj2, c2, 0), axis=0)
                      - jnp.sum(jnp.where(selj2, hr, 0), axis=0))
        return bstar, above_excl

    def do_row(r, buf):
        @pl.when(r + 1 < rows_per_w)
        def _():
            pltpu.async_copy(x_hbm.at[base + r + 1], xrow.at[1 - buf], sem_in)
        pltpu.make_async_copy(x_hbm.at[base + r], xrow.at[buf], sem_in).wait()

        def pp(j, _):
            v = xrow[buf, pl.ds(pl.multiple_of(j * _L, _L), _L)]
            a = plsc.bitcast(v, jnp.int32) & 0x7FFFFFFF
            col = ci + j * _L
            excl = (col >= wo) & (col < wo_hi)
            ab[pl.ds(pl.multiple_of(j * _L, _L), _L)] = jnp.where(excl, 0, a)
            return 0
        lax.fori_loop(0, nv, pp, 0, unroll=4)

        rk = jnp.int32(n_cand)
        prefix = jnp.int32(0)
        for sh, w in ((23, 8), (15, 8), (7, 8), (0, 7)):
            def cl(ii, _):
                hist[pl.ds(ii * _L, _L)] = zeros_i
                return 0
            lax.fori_loop(0, _L, cl, 0, unroll=4)

            def hp(j, _, sh=sh, w=w, prefix=prefix):
                a = ab[pl.ds(pl.multiple_of(j * _L, _L), _L)]
                act = lax.shift_right_logical(a, sh + w) == prefix
                b = lax.shift_right_logical(a, sh) & (2 ** w - 1)
                plsc.addupdate_scatter(hist, [b], ones_i, mask=act)
                return 0
            lax.fori_loop(0, nv, hp, 0, unroll=4)
            bstar, above = scan_hist(rk)
            rk = rk - above
            prefix = prefix * (2 ** w) + bstar
        tstar = prefix

        rvec = zeros_i + r

        def cb(j, run):
            a = ab[pl.ds(pl.multiple_of(j * _L, _L), _L)]
            msel = a >= tstar
            mi = jnp.where(msel, 1, 0)
            cs = plsc.cumsum(mi)
            posn = run + cs - mi
            smask = msel & (posn < n_cand)
            plsc.store_scatter(candbuf, [rvec, posn],
                               plsc.bitcast(a, jnp.float32), mask=smask)
            return run + jnp.sum(mi, axis=0)
        lax.fori_loop(0, nv, cb, jnp.int32(0), unroll=4)
        return ()

    def outer(g, _):
        do_row(g * 2, 0)
        do_row(g * 2 + 1, 1)
        return 0
    lax.fori_loop(0, rows_per_w // 2, outer, 0)

    pltpu.sync_copy(candbuf, out_hbm.at[pl.ds(obase, rows_per_w)])


def _mask_common(x_ref, os_ref, xw_ref):
    xr = x_ref[...]
    m = jnp.max(xr, axis=1, keepdims=True)
    e = jnp.exp(xr - m)
    s = jnp.sum(e, axis=1, keepdims=True)
    p = e / s
    ent = -jnp.sum(p * jnp.log(p + 1e-08), axis=1, keepdims=True)
    scaled = os_ref[...] * ent
    xw = xw_ref[...]
    xnw = xw + scaled
    abw = jax.lax.bitcast_convert_type(jnp.abs(xnw), jnp.int32)
    abwo = jax.lax.bitcast_convert_type(jnp.abs(xw), jnp.int32)
    abr = jax.lax.bitcast_convert_type(jnp.abs(xr), jnp.int32)
    return xr, xnw, abr, abwo, abw


def _mask_store(idx_ref, out_ref, xr, xnw, abr, abw, hi, k_dim):
    out_ref[...] = jnp.where(abr >= hi, xr, 0.0)
    wv = idx_ref[0] * k_dim
    out_ref[:, pl.ds(wv, k_dim)] = jnp.where(abw >= hi, xnw, 0.0)


def _mask_full_kernel(idx_ref, x_ref, os_ref, xw_ref, out_ref, *, sparsity_k):
    xr, xnw, abr, abwo, abw = _mask_common(x_ref, os_ref, xw_ref)
    r_dim = xr.shape[0]
    lo0 = jnp.full((r_dim, 1), -1, jnp.int32)
    hi0 = jnp.full((r_dim, 1), 0x7F800000, jnp.int32)

    def body(_, carry):
        lo, hi = carry
        mid = lo + jax.lax.shift_right_logical(hi - lo, 1)
        c1 = jnp.sum((abr > mid).astype(jnp.int32), axis=1, keepdims=True)
        c2 = jnp.sum((abwo > mid).astype(jnp.int32), axis=1, keepdims=True)
        c3 = jnp.sum((abw > mid).astype(jnp.int32), axis=1, keepdims=True)
        ge = (c1 - c2 + c3) >= sparsity_k
        return jnp.where(ge, mid, lo), jnp.where(ge, hi, mid)

    _, hi = jax.lax.fori_loop(0, 31, body, (lo0, hi0))
    _mask_store(idx_ref, out_ref, xr, xnw, abr, abw, hi, os_ref.shape[1])


def _mask_merge_kernel(idx_ref, canvas_ref, x_ref, os_ref, xw_ref, cand_ref,
                       out_ref, *, sparsity_k):
    del canvas_ref
    xr, xnw, abr, _, abw = _mask_common(x_ref, os_ref, xw_ref)
    r_dim = xr.shape[0]
    abc = jax.lax.bitcast_convert_type(cand_ref[...], jnp.int32)
    ab = jnp.concatenate([abw, abc], axis=1)
    lo0 = jnp.full((r_dim, 1), -1, jnp.int32)
    hi0 = jnp.full((r_dim, 1), 0x7F800000, jnp.int32)

    def body(_, carry):
        lo, hi = carry
        mid = lo + jax.lax.shift_right_logical(hi - lo, 1)
        cnt = jnp.sum((ab > mid).astype(jnp.int32), axis=1, keepdims=True)
        ge = cnt >= sparsity_k
        return jnp.where(ge, mid, lo), jnp.where(ge, hi, mid)

    _, hi = jax.lax.fori_loop(0, 31, body, (lo0, hi0))
    _mask_store(idx_ref, out_ref, xr, xnw, abr, abw, hi, os_ref.shape[1])


def kernel(x, Wq, Wk, Wv, Wo, decay_logit, out_scale, W_down, W_up, mlp_bias,
           mem_scale, write_scale, read_indices, write_selector):
    b_dim, t_dim, v_dim = x.shape
    k_dim = Wq.shape[0]
    inner = W_down.shape[0]
    eps = 1.1920929e-07
    rows = b_dim * t_dim

    scale = 1.0 / math.sqrt(k_dim)
    wq2 = Wq * scale
    wo2 = Wo * (out_scale * mem_scale[0])
    wup2 = W_up * (write_scale / (math.sqrt(k_dim) * math.log(v_dim)))
    log_decay = jnp.log(jax.nn.sigmoid(decay_logit)).reshape(1)
    ro_blk = (read_indices[0].astype(jnp.int32) // k_dim).reshape(1)
    wo_col = jnp.argmax(write_selector[0]).astype(jnp.int32)
    wo_blk = (wo_col // k_dim).reshape(1)

    x2 = x.reshape(rows, v_dim)

    rblk = min(_ROW_BLOCK, rows)
    n_blk = rows // rblk
    split_blk = max(n_blk - 1, 0)
    sc_rows = rows - split_blk * rblk
    row_base = split_blk * rblk

    rows_per_w = sc_rows // _NW
    mesh = plsc.VectorSubcoreMesh(core_axis_name="c", subcore_axis_name="s")
    sc_topk = functools.partial(
        pl.kernel,
        out_type=jax.ShapeDtypeStruct((sc_rows, _SPARSITY_K), jnp.float32),
        mesh=mesh,
        scratch_types=[
            pltpu.VMEM((2, v_dim), jnp.float32),
            pltpu.VMEM((v_dim,), jnp.int32),
            pltpu.VMEM((256,), jnp.int32),
            pltpu.VMEM((rows_per_w, _SPARSITY_K), jnp.float32),
            pltpu.VMEM((_L,), jnp.int32),
            pltpu.SemaphoreType.DMA,
        ],
        compiler_params=pltpu.CompilerParams(needs_layout_passes=False),
    )(functools.partial(_sc_topk_kernel, rows_per_w=rows_per_w, v_dim=v_dim,
                        n_cand=_SPARSITY_K, row_base=row_base))
    cands = sc_topk(x2, jnp.full((_L,), wo_col, jnp.int32))

    grid_a = pltpu.PrefetchScalarGridSpec(
        num_scalar_prefetch=1,
        grid=(b_dim,),
        in_specs=[
            pl.BlockSpec((1, t_dim, k_dim), lambda b, idx: (b, 0, idx[0])),
            pl.BlockSpec((k_dim, k_dim), lambda b, idx: (0, 0)),
            pl.BlockSpec((k_dim, k_dim), lambda b, idx: (0, 0)),
            pl.BlockSpec((k_dim, k_dim), lambda b, idx: (0, 0)),
            pl.BlockSpec((k_dim, k_dim), lambda b, idx: (0, 0)),
            pl.BlockSpec((inner, k_dim), lambda b, idx: (0, 0)),
            pl.BlockSpec((k_dim, inner), lambda b, idx: (0, 0)),
            pl.BlockSpec((1, inner), lambda b, idx: (0, 0)),
            pl.BlockSpec(memory_space=pltpu.SMEM),
        ],
        out_specs=pl.BlockSpec((1, t_dim, k_dim), lambda b, idx: (b, 0, 0)),
    )
    out_small = pl.pallas_call(
        functools.partial(_attn_mlp_kernel, eps=eps),
        grid_spec=grid_a,
        out_shape=jax.ShapeDtypeStruct((b_dim, t_dim, k_dim), jnp.float32),
    )(ro_blk, x, wq2, Wk, Wv, wo2, W_down, wup2,
      mlp_bias.reshape(1, inner), log_decay)

    os2 = out_small.reshape(rows, k_dim)
    grid_b1 = pltpu.PrefetchScalarGridSpec(
        num_scalar_prefetch=1,
        grid=(split_blk,),
        in_specs=[
            pl.BlockSpec((rblk, v_dim), lambda i, idx: (i, 0)),
            pl.BlockSpec((rblk, k_dim), lambda i, idx: (i, 0)),
            pl.BlockSpec((rblk, k_dim), lambda i, idx: (i, idx[0])),
        ],
        out_specs=pl.BlockSpec((rblk, v_dim), lambda i, idx: (i, 0)),
    )
    out1 = pl.pallas_call(
        functools.partial(_mask_full_kernel, sparsity_k=_SPARSITY_K),
        grid_spec=grid_b1,
        out_shape=jax.ShapeDtypeStruct((rows, v_dim), jnp.float32),
    )(wo_blk, x2, os2, x2)

    last = n_blk - 1
    grid_b2 = pltpu.PrefetchScalarGridSpec(
        num_scalar_prefetch=1,
        grid=(1,),
        in_specs=[
            pl.BlockSpec(memory_space=pl.ANY),
            pl.BlockSpec((rblk, v_dim), lambda i, idx: (last, 0)),
            pl.BlockSpec((rblk, k_dim), lambda i, idx: (last, 0)),
            pl.BlockSpec((rblk, k_dim), lambda i, idx: (last, idx[0])),
            pl.BlockSpec((rblk, _SPARSITY_K), lambda i, idx: (0, 0)),
        ],
        out_specs=pl.BlockSpec((rblk, v_dim), lambda i, idx: (last, 0)),
    )
    out = pl.pallas_call(
        functools.partial(_mask_merge_kernel, sparsity_k=_SPARSITY_K),
        grid_spec=grid_b2,
        out_shape=jax.ShapeDtypeStruct((rows, v_dim), jnp.float32),
        input_output_aliases={1: 0},
    )(wo_blk, out1, x2, os2, x2, cands)
    return out.reshape(b_dim, t_dim, v_dim)
